# baseline (speedup 1.0000x reference)
import functools

import jax
import jax.numpy as jnp
from jax.experimental import pallas as pl

N = 10000
D = 128
BM = 200


def _xw_body(x_ref, w_ref, out_ref):
    out_ref[...] = jnp.dot(x_ref[...], w_ref[...],
                           preferred_element_type=jnp.float32)


def _layer1_body(adj_ref, s1_ref, w2_ref, s2_ref):
    a = adj_ref[...]
    agg = jnp.dot(a, s1_ref[...], preferred_element_type=jnp.float32)
    rs = jnp.sum(a, axis=1, keepdims=True)
    h = jnp.maximum(agg / rs, 0.0)
    s2_ref[...] = jnp.dot(h, w2_ref[...], preferred_element_type=jnp.float32)


def _layer2_body(adj_ref, s2_ref, out_ref):
    a = adj_ref[...]
    agg = jnp.dot(a, s2_ref[...], preferred_element_type=jnp.float32)
    rs = jnp.sum(a, axis=1, keepdims=True)
    out_ref[...] = agg / rs


@jax.jit
def kernel(x, adjacency, W1, W2):
    support1 = pl.pallas_call(
        _xw_body,
        grid=(N // 2000,),
        in_specs=[
            pl.BlockSpec((2000, D), lambda i: (i, 0)),
            pl.BlockSpec((D, D), lambda i: (0, 0)),
        ],
        out_specs=pl.BlockSpec((2000, D), lambda i: (i, 0)),
        out_shape=jax.ShapeDtypeStruct((N, D), jnp.float32),
    )(x, W1)

    support2 = pl.pallas_call(
        _layer1_body,
        grid=(N // BM,),
        in_specs=[
            pl.BlockSpec((BM, N), lambda i: (i, 0)),
            pl.BlockSpec((N, D), lambda i: (0, 0)),
            pl.BlockSpec((D, D), lambda i: (0, 0)),
        ],
        out_specs=pl.BlockSpec((BM, D), lambda i: (i, 0)),
        out_shape=jax.ShapeDtypeStruct((N, D), jnp.float32),
    )(adjacency, support1, W2)

    logits = pl.pallas_call(
        _layer2_body,
        grid=(N // BM,),
        in_specs=[
            pl.BlockSpec((BM, N), lambda i: (i, 0)),
            pl.BlockSpec((N, D), lambda i: (0, 0)),
        ],
        out_specs=pl.BlockSpec((BM, D), lambda i: (i, 0)),
        out_shape=jax.ShapeDtypeStruct((N, D), jnp.float32),
    )(adjacency, support2)

    return logits


# speedup vs baseline: 1.0551x; 1.0551x over previous
import jax
import jax.numpy as jnp
from jax.experimental import pallas as pl
import jax.experimental.pallas.tpu as pltpu

N = 10000
D = 128
BM = 400
NI = N // BM
GRID = 2 * NI


def _xw_body(x_ref, w_ref, out_ref):
    out_ref[...] = jnp.dot(x_ref[...], w_ref[...],
                           preferred_element_type=jnp.float32)


def _sage_body(adj_ref, s1_ref, w2_ref, out_ref, s2_ref):
    i = pl.program_id(0)
    a = adj_ref[...]
    rs = jnp.sum(a, axis=1, keepdims=True)

    @pl.when(i < NI)
    def _():
        agg = jnp.dot(a, s1_ref[...], preferred_element_type=jnp.float32)
        h = jnp.maximum(agg / rs, 0.0)
        s2_ref[pl.ds(i * BM, BM), :] = jnp.dot(
            h, w2_ref[...], preferred_element_type=jnp.float32)

    @pl.when(i >= NI)
    def _():
        agg = jnp.dot(a, s2_ref[...], preferred_element_type=jnp.float32)
        out_ref[...] = agg / rs


def _adj_map(i):
    return (jnp.where(i < NI, i, GRID - 1 - i), 0)


def _out_map(i):
    return (jnp.where(i < NI, NI - 1, GRID - 1 - i), 0)


@jax.jit
def kernel(x, adjacency, W1, W2):
    support1 = pl.pallas_call(
        _xw_body,
        grid=(N // 2000,),
        in_specs=[
            pl.BlockSpec((2000, D), lambda i: (i, 0)),
            pl.BlockSpec((D, D), lambda i: (0, 0)),
        ],
        out_specs=pl.BlockSpec((2000, D), lambda i: (i, 0)),
        out_shape=jax.ShapeDtypeStruct((N, D), jnp.float32),
    )(x, W1)

    return pl.pallas_call(
        _sage_body,
        grid=(GRID,),
        in_specs=[
            pl.BlockSpec((BM, N), _adj_map),
            pl.BlockSpec((N, D), lambda i: (0, 0)),
            pl.BlockSpec((D, D), lambda i: (0, 0)),
        ],
        out_specs=pl.BlockSpec((BM, D), _out_map),
        out_shape=jax.ShapeDtypeStruct((N, D), jnp.float32),
        scratch_shapes=[
            pltpu.VMEM((N, D), jnp.float32),
        ],
    )(adjacency, support1, W2)


# speedup vs baseline: 1.0815x; 1.0250x over previous
import jax
import jax.numpy as jnp
from jax.experimental import pallas as pl
import jax.experimental.pallas.tpu as pltpu

N = 10000
D = 128
BM = 400
NI = N // BM
GRID = 2 * NI


def _sage_body(adj_ref, x_ref, w1_ref, w2_ref, out_ref, s2_ref):
    i = pl.program_id(0)
    a = adj_ref[...]
    rs = jnp.sum(a, axis=1, keepdims=True)

    @pl.when(i < NI)
    def _():
        ax = jnp.dot(a, x_ref[...], preferred_element_type=jnp.float32)
        agg = jnp.dot(ax, w1_ref[...], preferred_element_type=jnp.float32)
        h = jnp.maximum(agg / rs, 0.0)
        s2_ref[pl.ds(i * BM, BM), :] = jnp.dot(
            h, w2_ref[...], preferred_element_type=jnp.float32)

    @pl.when(i >= NI)
    def _():
        agg = jnp.dot(a, s2_ref[...], preferred_element_type=jnp.float32)
        out_ref[...] = agg / rs


def _adj_map(i):
    return (jnp.where(i < NI, i, GRID - 1 - i), 0)


def _out_map(i):
    return (jnp.where(i < NI, NI - 1, GRID - 1 - i), 0)


@jax.jit
def kernel(x, adjacency, W1, W2):
    return pl.pallas_call(
        _sage_body,
        grid=(GRID,),
        in_specs=[
            pl.BlockSpec((BM, N), _adj_map),
            pl.BlockSpec((N, D), lambda i: (0, 0)),
            pl.BlockSpec((D, D), lambda i: (0, 0)),
            pl.BlockSpec((D, D), lambda i: (0, 0)),
        ],
        out_specs=pl.BlockSpec((BM, D), _out_map),
        out_shape=jax.ShapeDtypeStruct((N, D), jnp.float32),
        scratch_shapes=[
            pltpu.VMEM((N, D), jnp.float32),
        ],
    )(adjacency, x, W1, W2)
